# Initial kernel scaffold; baseline (speedup 1.0000x reference)
import jax
import jax.numpy as jnp
from jax.experimental import pallas as pl


def kernel(x, edge_index, edge_attr_emb, v_idx, v_edge_emb, W1d, b1d, g1d, be1d, W2d, b2d, eps_d, W1u, b1u, g1u, be1u, W2u, b2u, eps_u, g_bn, be_bn, alpha1, alpha2):
    raise NotImplementedError("write your pallas kernel here")


# speedup vs baseline: 3.0432x; 3.0432x over previous
import functools

import jax
import jax.numpy as jnp
from jax import lax
from jax.experimental import pallas as pl
from jax.experimental.pallas import tpu as pltpu
from jax.experimental.pallas import tpu_sc as plsc

NC = 2
NS = 16
L = 16
K = 80


def _make_sc_agg(N, D, E, EV):
    npad = ((N + 8 * NS - 1) // (8 * NS)) * (8 * NS)
    rows_per_sub = npad // NS
    e_per_tile = E // (NC * NS)
    ev_per_tile = EV // (NC * NS)

    mesh = plsc.VectorSubcoreMesh(
        core_axis_name="c", subcore_axis_name="s",
        num_cores=NC, num_subcores=NS)

    @functools.partial(
        pl.kernel,
        out_type=(jax.ShapeDtypeStruct((NC, npad, D), jnp.float32),
                  jax.ShapeDtypeStruct((NC, npad, D), jnp.float32)),
        mesh=mesh,
        scratch_types=dict(
            src_v=pltpu.VMEM((K,), jnp.int32),
            dst_v=pltpu.VMEM((K,), jnp.int32),
            rows_v=pltpu.VMEM((K, D), jnp.float32),
            ea_v=pltpu.VMEM((K, D), jnp.float32),
            acc=pltpu.VMEM_SHARED((npad, D), jnp.float32),
            gsem=pltpu.SemaphoreType.DMA,
        ),
    )
    def agg(x_hbm, srcd, dstd, ead, srcu, dstu, eau, zeros_hbm,
            aggd, aggu, *, src_v, dst_v, rows_v, ea_v, acc, gsem):
        cid = lax.axis_index("c")
        sid = lax.axis_index("s")
        wid = sid * NC + cid
        r0 = sid * rows_per_sub

        def phase(src, dst, ea, out, per_tile):
            pltpu.sync_copy(zeros_hbm.at[pl.ds(r0, rows_per_sub)],
                            acc.at[pl.ds(r0, rows_per_sub)])
            plsc.subcore_barrier()
            base0 = wid * per_tile

            @pl.loop(0, per_tile // K)
            def _chunk(ci):
                base = base0 + ci * K
                pltpu.sync_copy(src.at[pl.ds(base, K)], src_v)
                pltpu.sync_copy(dst.at[pl.ds(base, K)], dst_v)
                pltpu.sync_copy(ea.at[pl.ds(base, K)], ea_v)
                pltpu.async_copy(x_hbm.at[src_v], rows_v, gsem).wait()

                @pl.loop(0, K)
                def _row(r):
                    for j in range(D // L):
                        s = pl.ds(j * L, L)
                        rows_v[r, s] = jnp.maximum(
                            rows_v[r, s] + ea_v[r, s], 0.0)

                pltpu.sync_copy(rows_v, acc.at[dst_v], add=True)

            plsc.subcore_barrier()
            pltpu.sync_copy(acc.at[pl.ds(r0, rows_per_sub)],
                            out.at[cid, pl.ds(r0, rows_per_sub)])

        phase(srcd, dstd, ead, aggd, e_per_tile)
        plsc.subcore_barrier()
        phase(srcu, dstu, eau, aggu, ev_per_tile)

    return agg


def _tc_tail(x_ref, aggd_ref, aggu_ref,
             W1d_ref, b1d_ref, g1d_ref, be1d_ref, W2d_ref, b2d_ref,
             W1u_ref, b1u_ref, g1u_ref, be1u_ref, W2u_ref, b2u_ref,
             g_bn_ref, be_bn_ref, scal_ref, out_ref):
    n = x_ref.shape[0]
    x = x_ref[...]
    eps_d = scal_ref[0]
    eps_u = scal_ref[1]
    a1 = scal_ref[2]
    a2 = scal_ref[3]

    def bn(h, g, b):
        mu = jnp.mean(h, axis=0, keepdims=True)
        var = jnp.mean((h - mu) * (h - mu), axis=0, keepdims=True)
        return (h - mu) * lax.rsqrt(var + 1e-5) * g + b

    def gine_tail(agg, eps, W1, b1, g1, be1, W2, b2):
        h = (1.0 + eps) * x + agg
        h = jnp.dot(h, W1, preferred_element_type=jnp.float32) + b1
        h = jnp.maximum(bn(h, g1, be1), 0.0)
        return jnp.dot(h, W2, preferred_element_type=jnp.float32) + b2

    hd = gine_tail(aggd_ref[0, :n, :] + aggd_ref[1, :n, :], eps_d,
                   W1d_ref[...], b1d_ref[...], g1d_ref[...], be1d_ref[...],
                   W2d_ref[...], b2d_ref[...])
    hu = gine_tail(aggu_ref[0, :n, :] + aggu_ref[1, :n, :], eps_u,
                   W1u_ref[...], b1u_ref[...], g1u_ref[...], be1u_ref[...],
                   W2u_ref[...], b2u_ref[...])
    out = x + a1 * hd + a2 * hu
    out_ref[...] = jnp.maximum(bn(out, g_bn_ref[...], be_bn_ref[...]), 0.0)


def kernel(x, edge_index, edge_attr_emb, v_idx, v_edge_emb,
           W1d, b1d, g1d, be1d, W2d, b2d, eps_d,
           W1u, b1u, g1u, be1u, W2u, b2u, eps_u,
           g_bn, be_bn, alpha1, alpha2):
    N, D = x.shape
    E = edge_attr_emb.shape[0]
    EV = v_edge_emb.shape[0]

    srcd = edge_index[0].astype(jnp.int32)
    dstd = edge_index[1].astype(jnp.int32)
    srcu = v_idx[0].astype(jnp.int32)
    dstu = v_idx[1].astype(jnp.int32)
    npad = ((N + 8 * NS - 1) // (8 * NS)) * (8 * NS)
    zeros = jnp.zeros((npad, D), jnp.float32)

    aggd, aggu = _make_sc_agg(N, D, E, EV)(
        x, srcd, dstd, edge_attr_emb, srcu, dstu, v_edge_emb, zeros)

    scal = jnp.stack([eps_d.astype(jnp.float32), eps_u.astype(jnp.float32),
                      alpha1.astype(jnp.float32), alpha2.astype(jnp.float32)])

    n_vmem = 17
    out = pl.pallas_call(
        _tc_tail,
        out_shape=jax.ShapeDtypeStruct((N, D), jnp.float32),
        in_specs=[pl.BlockSpec(memory_space=pltpu.VMEM)] * n_vmem
                 + [pl.BlockSpec(memory_space=pltpu.SMEM)],
        out_specs=pl.BlockSpec(memory_space=pltpu.VMEM),
    )(x, aggd, aggu,
      W1d, b1d, g1d, be1d, W2d, b2d,
      W1u, b1u, g1u, be1u, W2u, b2u,
      g_bn, be_bn, scal)
    return out


# speedup vs baseline: 6.2963x; 2.0690x over previous
import functools

import jax
import jax.numpy as jnp
from jax import lax
from jax.experimental import pallas as pl
from jax.experimental.pallas import tpu as pltpu
from jax.experimental.pallas import tpu_sc as plsc

NC = 2
NS = 16
L = 16
K = 80


def _make_sc_agg(N, D, E, EV):
    npad = ((N + 8 * NS - 1) // (8 * NS)) * (8 * NS)
    rows_per_sub = npad // NS
    e_per_tile = E // (NC * NS)
    ev_per_tile = EV // (NC * NS)

    mesh = plsc.VectorSubcoreMesh(
        core_axis_name="c", subcore_axis_name="s",
        num_cores=NC, num_subcores=NS)

    @functools.partial(
        pl.kernel,
        out_type=(jax.ShapeDtypeStruct((NC, npad, D), jnp.float32),
                  jax.ShapeDtypeStruct((NC, npad, D), jnp.float32)),
        mesh=mesh,
        scratch_types=dict(
            src_v=pltpu.VMEM((2, K), jnp.int32),
            dst_v=pltpu.VMEM((2, K), jnp.int32),
            rows_v=pltpu.VMEM((2, K, D), jnp.float32),
            ea_v=pltpu.VMEM((2, K, D), jnp.float32),
            acc=pltpu.VMEM_SHARED((npad, D), jnp.float32),
            isem=pltpu.SemaphoreType.DMA((2,)),
            esem=pltpu.SemaphoreType.DMA((2,)),
            gsem=pltpu.SemaphoreType.DMA((2,)),
            ssem=pltpu.SemaphoreType.DMA((2,)),
        ),
    )
    def agg(x_hbm, srcd, dstd, ead, srcu, dstu, eau, zeros_hbm,
            aggd, aggu, *, src_v, dst_v, rows_v, ea_v, acc,
            isem, esem, gsem, ssem):
        cid = lax.axis_index("c")
        sid = lax.axis_index("s")
        wid = sid * NC + cid
        r0 = sid * rows_per_sub

        def phase(src, dst, ea, out, per_tile):
            pltpu.sync_copy(zeros_hbm.at[pl.ds(r0, rows_per_sub)],
                            acc.at[pl.ds(r0, rows_per_sub)])
            plsc.subcore_barrier()
            base0 = wid * per_tile
            nch = per_tile // K

            def issue_idx(c, b):
                base = base0 + c * K
                pltpu.async_copy(src.at[pl.ds(base, K)], src_v.at[b],
                                 isem.at[b])
                pltpu.async_copy(dst.at[pl.ds(base, K)], dst_v.at[b],
                                 isem.at[b])
                pltpu.async_copy(ea.at[pl.ds(base, K)], ea_v.at[b],
                                 esem.at[b])

            def wait_idx(b):
                pltpu.make_async_copy(src.at[pl.ds(0, K)], src_v.at[b],
                                      isem.at[b]).wait()
                pltpu.make_async_copy(dst.at[pl.ds(0, K)], dst_v.at[b],
                                      isem.at[b]).wait()

            def issue_gather(b):
                pltpu.async_copy(x_hbm.at[src_v.at[b]], rows_v.at[b],
                                 gsem.at[b])

            def wait_rows(b):
                pltpu.make_async_copy(x_hbm.at[src_v.at[b]], rows_v.at[b],
                                      gsem.at[b]).wait()
                pltpu.make_async_copy(ea.at[pl.ds(0, K)], ea_v.at[b],
                                      esem.at[b]).wait()

            def issue_scatter(b):
                pltpu.async_copy(rows_v.at[b], acc.at[dst_v.at[b]],
                                 ssem.at[b], add=True)

            def wait_scatter(b):
                pltpu.make_async_copy(rows_v.at[b], acc.at[dst_v.at[b]],
                                      ssem.at[b]).wait()

            def compute(b):
                @pl.loop(0, K)
                def _row(r):
                    for j in range(D // L):
                        s = pl.ds(j * L, L)
                        rows_v[b, r, s] = jnp.maximum(
                            rows_v[b, r, s] + ea_v[b, r, s], 0.0)

            issue_idx(0, 0)
            wait_idx(0)
            issue_gather(0)

            npairs = (nch + 1) // 2
            @pl.loop(0, npairs)
            def _pair(p):
                for b in range(2):
                    nb = 1 - b
                    c = p * 2 + b

                    @pl.when(c < nch)
                    def _chunk():
                        @pl.when(c >= 1)
                        def _():
                            wait_scatter(nb)
                        @pl.when(c + 1 < nch)
                        def _():
                            issue_idx(c + 1, nb)
                            wait_idx(nb)
                            issue_gather(nb)
                        wait_rows(b)
                        compute(b)
                        issue_scatter(b)

            wait_scatter((nch - 1) % 2)
            plsc.subcore_barrier()
            pltpu.sync_copy(acc.at[pl.ds(r0, rows_per_sub)],
                            out.at[cid, pl.ds(r0, rows_per_sub)])

        phase(srcd, dstd, ead, aggd, e_per_tile)
        plsc.subcore_barrier()
        phase(srcu, dstu, eau, aggu, ev_per_tile)

    return agg


def _tc_tail(x_ref, aggd_ref, aggu_ref,
             W1d_ref, b1d_ref, g1d_ref, be1d_ref, W2d_ref, b2d_ref,
             W1u_ref, b1u_ref, g1u_ref, be1u_ref, W2u_ref, b2u_ref,
             g_bn_ref, be_bn_ref, scal_ref, out_ref):
    n = x_ref.shape[0]
    x = x_ref[...]
    eps_d = scal_ref[0]
    eps_u = scal_ref[1]
    a1 = scal_ref[2]
    a2 = scal_ref[3]

    def bn(h, g, b):
        mu = jnp.mean(h, axis=0, keepdims=True)
        var = jnp.mean((h - mu) * (h - mu), axis=0, keepdims=True)
        return (h - mu) * lax.rsqrt(var + 1e-5) * g + b

    def gine_tail(agg, eps, W1, b1, g1, be1, W2, b2):
        h = (1.0 + eps) * x + agg
        h = jnp.dot(h, W1, preferred_element_type=jnp.float32) + b1
        h = jnp.maximum(bn(h, g1, be1), 0.0)
        return jnp.dot(h, W2, preferred_element_type=jnp.float32) + b2

    hd = gine_tail(aggd_ref[0, :n, :] + aggd_ref[1, :n, :], eps_d,
                   W1d_ref[...], b1d_ref[...], g1d_ref[...], be1d_ref[...],
                   W2d_ref[...], b2d_ref[...])
    hu = gine_tail(aggu_ref[0, :n, :] + aggu_ref[1, :n, :], eps_u,
                   W1u_ref[...], b1u_ref[...], g1u_ref[...], be1u_ref[...],
                   W2u_ref[...], b2u_ref[...])
    out = x + a1 * hd + a2 * hu
    out_ref[...] = jnp.maximum(bn(out, g_bn_ref[...], be_bn_ref[...]), 0.0)


def kernel(x, edge_index, edge_attr_emb, v_idx, v_edge_emb,
           W1d, b1d, g1d, be1d, W2d, b2d, eps_d,
           W1u, b1u, g1u, be1u, W2u, b2u, eps_u,
           g_bn, be_bn, alpha1, alpha2):
    N, D = x.shape
    E = edge_attr_emb.shape[0]
    EV = v_edge_emb.shape[0]

    srcd = edge_index[0].astype(jnp.int32)
    dstd = edge_index[1].astype(jnp.int32)
    srcu = v_idx[0].astype(jnp.int32)
    dstu = v_idx[1].astype(jnp.int32)
    npad = ((N + 8 * NS - 1) // (8 * NS)) * (8 * NS)
    zeros = jnp.zeros((npad, D), jnp.float32)

    aggd, aggu = _make_sc_agg(N, D, E, EV)(
        x, srcd, dstd, edge_attr_emb, srcu, dstu, v_edge_emb, zeros)

    scal = jnp.stack([eps_d.astype(jnp.float32), eps_u.astype(jnp.float32),
                      alpha1.astype(jnp.float32), alpha2.astype(jnp.float32)])

    n_vmem = 17
    out = pl.pallas_call(
        _tc_tail,
        out_shape=jax.ShapeDtypeStruct((N, D), jnp.float32),
        in_specs=[pl.BlockSpec(memory_space=pltpu.VMEM)] * n_vmem
                 + [pl.BlockSpec(memory_space=pltpu.SMEM)],
        out_specs=pl.BlockSpec(memory_space=pltpu.VMEM),
    )(x, aggd, aggu,
      W1d, b1d, g1d, be1d, W2d, b2d,
      W1u, b1u, g1u, be1u, W2u, b2u,
      g_bn, be_bn, scal)
    return out


# speedup vs baseline: 7.3689x; 1.1704x over previous
import functools

import jax
import jax.numpy as jnp
from jax import lax
from jax.experimental import pallas as pl
from jax.experimental.pallas import tpu as pltpu
from jax.experimental.pallas import tpu_sc as plsc

NC = 2
NS = 16
L = 16
K = 80


def _make_sc_agg(N, D, E, EV):
    npad = ((N + 8 * NS - 1) // (8 * NS)) * (8 * NS)
    rows_per_sub = npad // NS
    e_per_tile = E // (NC * NS)
    ev_per_tile = EV // (NC * NS)

    mesh = plsc.VectorSubcoreMesh(
        core_axis_name="c", subcore_axis_name="s",
        num_cores=NC, num_subcores=NS)

    nch_d = e_per_tile // K
    nch_u = ev_per_tile // K
    NB = 4

    @functools.partial(
        pl.kernel,
        out_type=(jax.ShapeDtypeStruct((NC, npad, D), jnp.float32),
                  jax.ShapeDtypeStruct((NC, npad, D), jnp.float32)),
        mesh=mesh,
        scratch_types=dict(
            src_v=pltpu.VMEM((NB, K), jnp.int32),
            dst_v=pltpu.VMEM((NB, K), jnp.int32),
            msg_v=pltpu.VMEM((NB, K, D), jnp.float32),
            acc=pltpu.VMEM_SHARED((npad, D), jnp.float32),
            isem=pltpu.SemaphoreType.DMA((NB,)),
            esem=pltpu.SemaphoreType.DMA((NB,)),
            gsem=pltpu.SemaphoreType.DMA((NB,)),
            ssem=pltpu.SemaphoreType.DMA((NB,)),
        ),
    )
    def agg(x_hbm, srcd, dstd, ead, srcu, dstu, eau, zeros_hbm,
            aggd, aggu, *, src_v, dst_v, msg_v, acc,
            isem, esem, gsem, ssem):
        cid = lax.axis_index("c")
        sid = lax.axis_index("s")
        wid = sid * NC + cid
        r0 = sid * rows_per_sub

        def phase(src, dst, ea, out, nch):
            pltpu.sync_copy(zeros_hbm.at[pl.ds(r0, rows_per_sub)],
                            acc.at[pl.ds(r0, rows_per_sub)])
            plsc.subcore_barrier()
            base0 = wid * nch * K

            def issue_idx_ea(c, b):
                pltpu.async_copy(src.at[wid, c], src_v.at[b], isem.at[b])
                pltpu.async_copy(dst.at[wid, c], dst_v.at[b], isem.at[b])
                pltpu.async_copy(ea.at[pl.ds(base0 + c * K, K)],
                                 msg_v.at[b], esem.at[b])

            def wait_idx(b):
                pltpu.make_async_copy(src.at[wid, 0], src_v.at[b],
                                      isem.at[b]).wait()
                pltpu.make_async_copy(dst.at[wid, 0], dst_v.at[b],
                                      isem.at[b]).wait()

            def wait_ea(b):
                pltpu.make_async_copy(ea.at[pl.ds(0, K)], msg_v.at[b],
                                      esem.at[b]).wait()

            def issue_gather_add(b):
                pltpu.async_copy(x_hbm.at[src_v.at[b]], msg_v.at[b],
                                 gsem.at[b], add=True)

            def wait_gather(b):
                pltpu.make_async_copy(x_hbm.at[src_v.at[0]], msg_v.at[b],
                                      gsem.at[b]).wait()

            def issue_scatter(b):
                pltpu.async_copy(msg_v.at[b], acc.at[dst_v.at[b]],
                                 ssem.at[b], add=True)

            def wait_scatter(b):
                pltpu.make_async_copy(msg_v.at[b], acc.at[dst_v.at[b]],
                                      ssem.at[b]).wait()

            def relu(b):
                @pl.loop(0, K)
                def _row(r):
                    for j in range(D // L):
                        s = pl.ds(j * L, L)
                        msg_v[b, r, s] = jnp.maximum(msg_v[b, r, s], 0.0)

            issue_idx_ea(0, 0)
            if nch > 1:
                issue_idx_ea(1, 1)
            wait_idx(0)
            wait_ea(0)
            issue_gather_add(0)

            ngroups = (nch + NB - 1) // NB
            @pl.loop(0, ngroups)
            def _group(p):
                for b in range(NB):
                    s1 = (b + 1) % NB
                    s2 = (b + 2) % NB
                    c = p * NB + b

                    @pl.when(c < nch)
                    def _chunk():
                        @pl.when(c >= 2)
                        def _():
                            wait_scatter(s2)
                        @pl.when(c + 2 < nch)
                        def _():
                            issue_idx_ea(c + 2, s2)
                        @pl.when(c + 1 < nch)
                        def _():
                            wait_idx(s1)
                            wait_ea(s1)
                            issue_gather_add(s1)
                        wait_gather(b)
                        relu(b)
                        issue_scatter(b)

            wait_scatter((nch - 1) % NB)
            if nch > 1:
                wait_scatter((nch - 2) % NB)
            plsc.subcore_barrier()
            pltpu.sync_copy(acc.at[pl.ds(r0, rows_per_sub)],
                            out.at[cid, pl.ds(r0, rows_per_sub)])

        phase(srcd, dstd, ead, aggd, nch_d)
        plsc.subcore_barrier()
        phase(srcu, dstu, eau, aggu, nch_u)

    return agg


def _tc_tail(x_ref, aggd_ref, aggu_ref,
             W1d_ref, b1d_ref, g1d_ref, be1d_ref, W2d_ref, b2d_ref,
             W1u_ref, b1u_ref, g1u_ref, be1u_ref, W2u_ref, b2u_ref,
             g_bn_ref, be_bn_ref, scal_ref, out_ref):
    n = x_ref.shape[0]
    x = x_ref[...]
    eps_d = scal_ref[0]
    eps_u = scal_ref[1]
    a1 = scal_ref[2]
    a2 = scal_ref[3]

    def bn(h, g, b):
        mu = jnp.mean(h, axis=0, keepdims=True)
        var = jnp.mean((h - mu) * (h - mu), axis=0, keepdims=True)
        return (h - mu) * lax.rsqrt(var + 1e-5) * g + b

    def gine_tail(agg, eps, W1, b1, g1, be1, W2, b2):
        h = (1.0 + eps) * x + agg
        h = jnp.dot(h, W1, preferred_element_type=jnp.float32) + b1
        h = jnp.maximum(bn(h, g1, be1), 0.0)
        return jnp.dot(h, W2, preferred_element_type=jnp.float32) + b2

    hd = gine_tail(aggd_ref[0, :n, :] + aggd_ref[1, :n, :], eps_d,
                   W1d_ref[...], b1d_ref[...], g1d_ref[...], be1d_ref[...],
                   W2d_ref[...], b2d_ref[...])
    hu = gine_tail(aggu_ref[0, :n, :] + aggu_ref[1, :n, :], eps_u,
                   W1u_ref[...], b1u_ref[...], g1u_ref[...], be1u_ref[...],
                   W2u_ref[...], b2u_ref[...])
    out = x + a1 * hd + a2 * hu
    out_ref[...] = jnp.maximum(bn(out, g_bn_ref[...], be_bn_ref[...]), 0.0)


def kernel(x, edge_index, edge_attr_emb, v_idx, v_edge_emb,
           W1d, b1d, g1d, be1d, W2d, b2d, eps_d,
           W1u, b1u, g1u, be1u, W2u, b2u, eps_u,
           g_bn, be_bn, alpha1, alpha2):
    N, D = x.shape
    E = edge_attr_emb.shape[0]
    EV = v_edge_emb.shape[0]

    nw = NC * NS
    srcd = edge_index[0].astype(jnp.int32).reshape(nw, -1, K)
    dstd = edge_index[1].astype(jnp.int32).reshape(nw, -1, K)
    srcu = v_idx[0].astype(jnp.int32).reshape(nw, -1, K)
    dstu = v_idx[1].astype(jnp.int32).reshape(nw, -1, K)
    npad = ((N + 8 * NS - 1) // (8 * NS)) * (8 * NS)
    zeros = jnp.zeros((npad, D), jnp.float32)

    aggd, aggu = _make_sc_agg(N, D, E, EV)(
        x, srcd, dstd, edge_attr_emb, srcu, dstu, v_edge_emb, zeros)

    scal = jnp.stack([eps_d.astype(jnp.float32), eps_u.astype(jnp.float32),
                      alpha1.astype(jnp.float32), alpha2.astype(jnp.float32)])

    n_vmem = 17
    out = pl.pallas_call(
        _tc_tail,
        out_shape=jax.ShapeDtypeStruct((N, D), jnp.float32),
        in_specs=[pl.BlockSpec(memory_space=pltpu.VMEM)] * n_vmem
                 + [pl.BlockSpec(memory_space=pltpu.SMEM)],
        out_specs=pl.BlockSpec(memory_space=pltpu.VMEM),
    )(x, aggd, aggu,
      W1d, b1d, g1d, be1d, W2d, b2d,
      W1u, b1u, g1u, be1u, W2u, b2u,
      g_bn, be_bn, scal)
    return out


# speedup vs baseline: 7.7313x; 1.0492x over previous
import functools

import jax
import jax.numpy as jnp
from jax import lax
from jax.experimental import pallas as pl
from jax.experimental.pallas import tpu as pltpu
from jax.experimental.pallas import tpu_sc as plsc

NC = 2
NS = 16
L = 16
K = 80


def _make_sc_agg(N, D, E, EV):
    npad = ((N + 8 * NS - 1) // (8 * NS)) * (8 * NS)
    rows_per_sub = npad // NS
    e_per_tile = E // (NC * NS)
    ev_per_tile = EV // (NC * NS)

    mesh = plsc.VectorSubcoreMesh(
        core_axis_name="c", subcore_axis_name="s",
        num_cores=NC, num_subcores=NS)

    nch_d = e_per_tile // K
    nch_u = ev_per_tile // K
    NB = 4

    @functools.partial(
        pl.kernel,
        out_type=(jax.ShapeDtypeStruct((NC, npad, D), jnp.float32),
                  jax.ShapeDtypeStruct((NC, npad, D), jnp.float32)),
        mesh=mesh,
        scratch_types=dict(
            src_v=pltpu.VMEM((NB, K), jnp.int32),
            dst_v=pltpu.VMEM((NB, K), jnp.int32),
            msg_v=pltpu.VMEM((NB, K, D), jnp.float32),
            acc=pltpu.VMEM_SHARED((npad, D), jnp.float32),
            isem=pltpu.SemaphoreType.DMA((NB,)),
            esem=pltpu.SemaphoreType.DMA((NB,)),
            gsem=pltpu.SemaphoreType.DMA((NB,)),
            ssem=pltpu.SemaphoreType.DMA((NB,)),
        ),
    )
    def agg(x_hbm, srcd, dstd, ead, srcu, dstu, eau,
            aggd, aggu, *, src_v, dst_v, msg_v, acc,
            isem, esem, gsem, ssem):
        cid = lax.axis_index("c")
        sid = lax.axis_index("s")
        wid = sid * NC + cid
        r0 = sid * rows_per_sub

        def phase(src, dst, ea, out, nch):
            @pl.loop(0, K)
            def _zrow(r):
                for j in range(D // L):
                    msg_v[0, r, pl.ds(j * L, L)] = jnp.zeros((L,), jnp.float32)
            for k in range(rows_per_sub // K):
                pltpu.sync_copy(msg_v.at[0],
                                acc.at[pl.ds(r0 + k * K, K)])
            plsc.subcore_barrier()
            base0 = wid * nch * K

            def issue_idx_ea(c, b):
                base = base0 + c * K
                pltpu.async_copy(src.at[pl.ds(base, K)], src_v.at[b],
                                 isem.at[b])
                pltpu.async_copy(dst.at[pl.ds(base, K)], dst_v.at[b],
                                 isem.at[b])
                pltpu.async_copy(ea.at[pl.ds(base, K)], msg_v.at[b],
                                 esem.at[b])

            def wait_idx(b):
                pltpu.make_async_copy(src.at[pl.ds(0, K)], src_v.at[b],
                                      isem.at[b]).wait()
                pltpu.make_async_copy(dst.at[pl.ds(0, K)], dst_v.at[b],
                                      isem.at[b]).wait()

            def wait_ea(b):
                pltpu.make_async_copy(ea.at[pl.ds(0, K)], msg_v.at[b],
                                      esem.at[b]).wait()

            def issue_gather_add(b):
                pltpu.async_copy(x_hbm.at[src_v.at[b]], msg_v.at[b],
                                 gsem.at[b], add=True)

            def wait_gather(b):
                pltpu.make_async_copy(x_hbm.at[src_v.at[0]], msg_v.at[b],
                                      gsem.at[b]).wait()

            def issue_scatter(b):
                pltpu.async_copy(msg_v.at[b], acc.at[dst_v.at[b]],
                                 ssem.at[b], add=True)

            def wait_scatter(b):
                pltpu.make_async_copy(msg_v.at[b], acc.at[dst_v.at[b]],
                                      ssem.at[b]).wait()

            def relu(b):
                @pl.loop(0, K)
                def _row(r):
                    for j in range(D // L):
                        s = pl.ds(j * L, L)
                        msg_v[b, r, s] = jnp.maximum(msg_v[b, r, s], 0.0)

            issue_idx_ea(0, 0)
            if nch > 1:
                issue_idx_ea(1, 1)
            wait_idx(0)
            wait_ea(0)
            issue_gather_add(0)

            ngroups = (nch + NB - 1) // NB
            @pl.loop(0, ngroups)
            def _group(p):
                for b in range(NB):
                    s1 = (b + 1) % NB
                    s2 = (b + 2) % NB
                    c = p * NB + b

                    @pl.when(c < nch)
                    def _chunk():
                        @pl.when(c >= 2)
                        def _():
                            wait_scatter(s2)
                        @pl.when(c + 2 < nch)
                        def _():
                            issue_idx_ea(c + 2, s2)
                        @pl.when(c + 1 < nch)
                        def _():
                            wait_idx(s1)
                            wait_ea(s1)
                            issue_gather_add(s1)
                        wait_gather(b)
                        relu(b)
                        issue_scatter(b)

            wait_scatter((nch - 1) % NB)
            if nch > 1:
                wait_scatter((nch - 2) % NB)
            plsc.subcore_barrier()
            pltpu.sync_copy(acc.at[pl.ds(r0, rows_per_sub)],
                            out.at[cid, pl.ds(r0, rows_per_sub)])

        phase(srcd, dstd, ead, aggd, nch_d)
        plsc.subcore_barrier()
        phase(srcu, dstu, eau, aggu, nch_u)

    return agg


def _tc_tail(x_ref, aggd_ref, aggu_ref,
             W1d_ref, b1d_ref, g1d_ref, be1d_ref, W2d_ref, b2d_ref,
             W1u_ref, b1u_ref, g1u_ref, be1u_ref, W2u_ref, b2u_ref,
             g_bn_ref, be_bn_ref, scal_ref, out_ref):
    n = x_ref.shape[0]
    x = x_ref[...]
    eps_d = scal_ref[0]
    eps_u = scal_ref[1]
    a1 = scal_ref[2]
    a2 = scal_ref[3]

    def bn(h, g, b):
        mu = jnp.mean(h, axis=0, keepdims=True)
        var = jnp.mean((h - mu) * (h - mu), axis=0, keepdims=True)
        return (h - mu) * lax.rsqrt(var + 1e-5) * g + b

    def gine_tail(agg, eps, W1, b1, g1, be1, W2, b2):
        h = (1.0 + eps) * x + agg
        h = jnp.dot(h, W1, preferred_element_type=jnp.float32) + b1
        h = jnp.maximum(bn(h, g1, be1), 0.0)
        return jnp.dot(h, W2, preferred_element_type=jnp.float32) + b2

    hd = gine_tail(aggd_ref[0, :n, :] + aggd_ref[1, :n, :], eps_d,
                   W1d_ref[...], b1d_ref[...], g1d_ref[...], be1d_ref[...],
                   W2d_ref[...], b2d_ref[...])
    hu = gine_tail(aggu_ref[0, :n, :] + aggu_ref[1, :n, :], eps_u,
                   W1u_ref[...], b1u_ref[...], g1u_ref[...], be1u_ref[...],
                   W2u_ref[...], b2u_ref[...])
    out = x + a1 * hd + a2 * hu
    out_ref[...] = jnp.maximum(bn(out, g_bn_ref[...], be_bn_ref[...]), 0.0)


def kernel(x, edge_index, edge_attr_emb, v_idx, v_edge_emb,
           W1d, b1d, g1d, be1d, W2d, b2d, eps_d,
           W1u, b1u, g1u, be1u, W2u, b2u, eps_u,
           g_bn, be_bn, alpha1, alpha2):
    N, D = x.shape
    E = edge_attr_emb.shape[0]
    EV = v_edge_emb.shape[0]

    srcd = edge_index[0].astype(jnp.int32)
    dstd = edge_index[1].astype(jnp.int32)
    srcu = v_idx[0].astype(jnp.int32)
    dstu = v_idx[1].astype(jnp.int32)
    aggd, aggu = _make_sc_agg(N, D, E, EV)(
        x, srcd, dstd, edge_attr_emb, srcu, dstu, v_edge_emb)

    scal = jnp.stack([eps_d.astype(jnp.float32), eps_u.astype(jnp.float32),
                      alpha1.astype(jnp.float32), alpha2.astype(jnp.float32)])

    n_vmem = 17
    out = pl.pallas_call(
        _tc_tail,
        out_shape=jax.ShapeDtypeStruct((N, D), jnp.float32),
        in_specs=[pl.BlockSpec(memory_space=pltpu.VMEM)] * n_vmem
                 + [pl.BlockSpec(memory_space=pltpu.SMEM)],
        out_specs=pl.BlockSpec(memory_space=pltpu.VMEM),
    )(x, aggd, aggu,
      W1d, b1d, g1d, be1d, W2d, b2d,
      W1u, b1u, g1u, be1u, W2u, b2u,
      g_bn, be_bn, scal)
    return out
